# baseline (speedup 1.0000x reference)
import functools

import jax
import jax.numpy as jnp
from jax import lax
from jax.experimental import pallas as pl
from jax.experimental.pallas import tpu as pltpu
from jax.experimental.pallas import tpu_sc as plsc

B, D, H, W = 8, 256, 24, 24
T = H * W
N = B * T
V = 8192
KT = 512
NK = V // KT

_BIG = 2**30


def _argmin_body(h_ref, cb_ref, fn_ref, cn_ref, out_ref, rmin_ref):
    k = pl.program_id(1)
    hb = h_ref[0]
    cbk = cb_ref[...]
    mm = lax.dot_general(cbk, hb, (((1,), (0,)), ((), ())),
                         preferred_element_type=jnp.float32)
    fn = fn_ref[0]
    cn = cn_ref[0]
    d2 = (fn - 2.0 * mm) + cn

    lmin = jnp.min(d2, axis=0, keepdims=True)
    iot = lax.broadcasted_iota(jnp.int32, (KT, T), 0)
    lidx = jnp.min(jnp.where(d2 == lmin, iot, _BIG),
                   axis=0, keepdims=True) + k * KT

    @pl.when(k == 0)
    def _init():
        rmin_ref[...] = lmin
        out_ref[0] = lidx

    @pl.when(k > 0)
    def _update():
        upd = lmin < rmin_ref[...]
        rmin_ref[...] = jnp.where(upd, lmin, rmin_ref[...])
        out_ref[0] = jnp.where(upd, lidx, out_ref[0])


def _nearest_codes(h3, cb, fn, cn):
    return pl.pallas_call(
        _argmin_body,
        grid=(B, NK),
        in_specs=[
            pl.BlockSpec((1, D, T), lambda b, k: (b, 0, 0)),
            pl.BlockSpec((KT, D), lambda b, k: (k, 0)),
            pl.BlockSpec((1, 1, T), lambda b, k: (b, 0, 0)),
            pl.BlockSpec((1, KT, 1), lambda b, k: (k, 0, 0)),
        ],
        out_specs=pl.BlockSpec((1, 1, T), lambda b, k: (b, 0, 0)),
        out_shape=jax.ShapeDtypeStruct((B, 1, T), jnp.int32),
        scratch_shapes=[pltpu.VMEM((1, T), jnp.float32)],
        compiler_params=pltpu.CompilerParams(
            dimension_semantics=("parallel", "arbitrary")),
    )(h3, cb, fn, cn)


_NC = 2
_NS = 16
_NW = _NC * _NS
_ROWS_PER_W = N // _NW
_CHUNK = 72
_NCHUNK = _ROWS_PER_W // _CHUNK


def _gather_body(idx_hbm, cb_hbm, out_hbm, idx_v, rows_v, sem):
    wid = lax.axis_index("s") * _NC + lax.axis_index("c")
    base = wid * _ROWS_PER_W
    pltpu.sync_copy(idx_hbm.at[pl.ds(wid * _NCHUNK, _NCHUNK)], idx_v)
    for c in range(_NCHUNK):
        pltpu.async_copy(cb_hbm.at[idx_v.at[c]], rows_v, sem).wait()
        pltpu.sync_copy(rows_v, out_hbm.at[pl.ds(base + c * _CHUNK, _CHUNK)])


def _gather_rows(idx2d, cb):
    return pl.kernel(
        _gather_body,
        mesh=plsc.VectorSubcoreMesh(core_axis_name="c", subcore_axis_name="s"),
        out_type=jax.ShapeDtypeStruct((N, D), jnp.float32),
        scratch_types=[
            pltpu.VMEM((_NCHUNK, _CHUNK), jnp.int32),
            pltpu.VMEM((_CHUNK, D), jnp.float32),
            pltpu.SemaphoreType.DMA,
        ],
    )(idx2d, cb)


def kernel(h, cb):
    h3 = h.reshape(B, D, T)
    flat = jnp.transpose(h, (0, 2, 3, 1)).reshape(N, D)
    fn = jnp.sum(flat * flat, axis=1).reshape(B, 1, T)
    cn = jnp.sum(cb * cb, axis=1).reshape(NK, KT, 1)
    idx = _nearest_codes(h3, cb, fn, cn)
    idx_flat = idx.reshape(N)
    q = _gather_rows(idx_flat.reshape(_NW * _NCHUNK, _CHUNK), cb)
    z = idx.reshape(B, H, W)
    return (z, q.reshape(B, H, W, D))


# speedup vs baseline: 1.7619x; 1.7619x over previous
import jax
import jax.numpy as jnp
from jax import lax
from jax.experimental import pallas as pl
from jax.experimental.pallas import tpu as pltpu
from jax.experimental.pallas import tpu_sc as plsc

B, D, H, W = 8, 256, 24, 24
N = B * H * W
V = 8192
KT = 512
NK = V // KT
TT = 512
NT = N // TT

_BIG = float(2**30)


def _argmin_body(ht_ref, cb_ref, fn_ref, cn_ref, out_ref):
    ht = ht_ref[...]
    fn = fn_ref[0]
    rmin = None
    ridx = None
    for k in range(NK):
        cbk = cb_ref[pl.ds(k * KT, KT), :]
        mm = lax.dot_general(cbk, ht, (((1,), (0,)), ((), ())),
                             preferred_element_type=jnp.float32)
        cn = cn_ref[pl.ds(k * KT, KT), :]
        d2 = (fn - 2.0 * mm) + cn
        lmin = jnp.min(d2, axis=0, keepdims=True)
        iot = lax.broadcasted_iota(jnp.int32, (KT, TT), 0).astype(jnp.float32)
        lidx = jnp.min(jnp.where(d2 == lmin, iot, _BIG),
                       axis=0, keepdims=True) + float(k * KT)
        if k == 0:
            rmin, ridx = lmin, lidx
        else:
            upd = lmin < rmin
            rmin = jnp.where(upd, lmin, rmin)
            ridx = jnp.where(upd, lidx, ridx)
    out_ref[0] = ridx.astype(jnp.int32)


def _nearest_codes(ht, cb, fn, cn):
    return pl.pallas_call(
        _argmin_body,
        grid=(NT,),
        in_specs=[
            pl.BlockSpec((D, TT), lambda t: (0, t)),
            pl.BlockSpec((V, D), lambda t: (0, 0)),
            pl.BlockSpec((1, 1, TT), lambda t: (t, 0, 0)),
            pl.BlockSpec((V, 1), lambda t: (0, 0)),
        ],
        out_specs=pl.BlockSpec((1, 1, TT), lambda t: (t, 0, 0)),
        out_shape=jax.ShapeDtypeStruct((NT, 1, TT), jnp.int32),
        compiler_params=pltpu.CompilerParams(
            dimension_semantics=("arbitrary",)),
    )(ht, cb, fn, cn)


_NC = 2
_NS = 16
_NW = _NC * _NS
_ROWS_PER_W = N // _NW
_CHUNK = 72
_NCHUNK = _ROWS_PER_W // _CHUNK


def _gather_body(idx_hbm, cb_hbm, out_hbm, idx_v, rows_v, sem):
    wid = lax.axis_index("s") * _NC + lax.axis_index("c")
    base = wid * _ROWS_PER_W
    pltpu.sync_copy(idx_hbm.at[pl.ds(wid * _NCHUNK, _NCHUNK)], idx_v)
    for c in range(_NCHUNK):
        pltpu.async_copy(cb_hbm.at[idx_v.at[c]], rows_v, sem).wait()
        pltpu.sync_copy(rows_v, out_hbm.at[pl.ds(base + c * _CHUNK, _CHUNK)])


def _gather_rows(idx2d, cb):
    return pl.kernel(
        _gather_body,
        mesh=plsc.VectorSubcoreMesh(core_axis_name="c", subcore_axis_name="s"),
        out_type=jax.ShapeDtypeStruct((N, D), jnp.float32),
        scratch_types=[
            pltpu.VMEM((_NCHUNK, _CHUNK), jnp.int32),
            pltpu.VMEM((_CHUNK, D), jnp.float32),
            pltpu.SemaphoreType.DMA,
        ],
    )(idx2d, cb)


def kernel(h, cb):
    flat = jnp.transpose(h, (0, 2, 3, 1)).reshape(N, D)
    ht = jnp.transpose(h.reshape(B, D, H * W), (1, 0, 2)).reshape(D, N)
    fn = jnp.sum(flat * flat, axis=1).reshape(NT, 1, TT)
    cn = jnp.sum(cb * cb, axis=1).reshape(V, 1)
    idx = _nearest_codes(ht, cb, fn, cn)
    idx_flat = idx.reshape(N)
    q = _gather_rows(idx_flat.reshape(_NW * _NCHUNK, _CHUNK), cb)
    z = idx_flat.reshape(B, H, W)
    return (z, q.reshape(B, H, W, D))
